# baseline (speedup 1.0000x reference)
import math

import jax
import jax.numpy as jnp
from jax.experimental import pallas as pl

N = 10000
NE = 320000
D = 128
H = 4
DK = D // H


def _copy_body(x_ref, o_ref):
    o_ref[...] = x_ref[...]


def kernel(h, e, edge_index, Wq, Wk, Wv, We, Wlin, blin, g1, b1,
           Wf1, bf1, Wf2, bf2, g2, b2):
    src = edge_index[0]
    dst = edge_index[1]
    Q = (h @ Wq).reshape(N, H, DK)
    K = (h @ Wk).reshape(N, H, DK)
    V = (h @ Wv).reshape(N, H, DK)
    Ee = (e @ We).reshape(NE, H, DK)
    score = (K[src] * Q[dst]).sum(-1, keepdims=True) / math.sqrt(DK)
    score = score * (1.0 + Ee)
    alpha = jnp.exp(score)
    Z = jnp.clip(jax.ops.segment_sum(alpha, dst, num_segments=N), 1e-6, None)
    m = V[src] * alpha
    h_out = jax.ops.segment_sum(m, dst, num_segments=N) / Z
    h_out = h_out.reshape(N, D)
    hr = h + (h_out @ Wlin + blin)
    mu = hr.mean(-1, keepdims=True)
    var = hr.var(-1, keepdims=True)
    hr = (hr - mu) / jnp.sqrt(var + 1e-5) * g1 + b1
    h2 = jax.nn.relu(hr @ Wf1 + bf1) @ Wf2 + bf2
    x = hr + h2
    mu = x.mean(-1, keepdims=True)
    var = x.var(-1, keepdims=True)
    out = (x - mu) / jnp.sqrt(var + 1e-5) * g2 + b2
    return pl.pallas_call(
        _copy_body,
        grid=(N // 1000,),
        in_specs=[pl.BlockSpec((1000, D), lambda i: (i, 0))],
        out_specs=pl.BlockSpec((1000, D), lambda i: (i, 0)),
        out_shape=jax.ShapeDtypeStruct((N, D), jnp.float32),
    )(out)


# speedup vs baseline: 37.4225x; 37.4225x over previous
import dataclasses
import functools
import math

import jax
import jax.numpy as jnp
from jax import lax
from jax.experimental import pallas as pl
from jax.experimental.pallas import tpu as pltpu
from jax.experimental.pallas import tpu_sc as plsc

N = 10000
NE = 320000
D = 128
H = 4
DK = D // H
INV_SQRT_DK = 1.0 / math.sqrt(DK)

BN = 1000
BE = 2000

NSC = 2
NSUB = 16
EPT = NE // NSUB
CB = 80
NCH = EPT // CB
NP = 10240
NROW = NP // NSUB


def _qkv_body(h_ref, wq_ref, wk_ref, wv_ref, q_ref, k_ref, v_ref):
    hb = h_ref[...]
    for w_ref, o_ref in ((wq_ref, q_ref), (wk_ref, k_ref), (wv_ref, v_ref)):
        ob = jnp.dot(hb, w_ref[...], preferred_element_type=jnp.float32)
        o_ref[0] = ob[:, :64]
        o_ref[1] = ob[:, 64:]


def _qkv(h, Wq, Wk, Wv):
    out = [jax.ShapeDtypeStruct((2, N, 64), jnp.float32) for _ in range(3)]
    wspec = lambda: pl.BlockSpec((D, D), lambda i: (0, 0))
    ospec = lambda: pl.BlockSpec((2, BN, 64), lambda i: (0, i, 0))
    return pl.pallas_call(
        _qkv_body,
        grid=(N // BN,),
        in_specs=[pl.BlockSpec((BN, D), lambda i: (i, 0)),
                  wspec(), wspec(), wspec()],
        out_specs=[ospec(), ospec(), ospec()],
        out_shape=out,
    )(h, Wq, Wk, Wv)


def _ee_body(e_ref, we_ref, o_ref):
    eb = jnp.dot(e_ref[...], we_ref[...], preferred_element_type=jnp.float32)
    o_ref[0] = eb[:, :64]
    o_ref[1] = eb[:, 64:]


def _ee(e, We):
    return pl.pallas_call(
        _ee_body,
        grid=(NE // BE,),
        in_specs=[pl.BlockSpec((BE, D), lambda i: (i, 0)),
                  pl.BlockSpec((D, D), lambda i: (0, 0))],
        out_specs=pl.BlockSpec((2, BE, 64), lambda i: (0, i, 0)),
        out_shape=jax.ShapeDtypeStruct((2, NE, 64), jnp.float32),
    )(e, We)


def _edge_body(kf, qf, vf, eef, src_hbm, dst_hbm, z_hbm, s_hbm,
               srcv, dstv, srcoff, dstoff, kg, qg, vg, eev,
               astage, mstage, zsh, ssh):
    c = lax.axis_index("c")
    s = lax.axis_index("s")

    zvec = jnp.zeros((16,), jnp.float32)

    @pl.loop(0, CB)
    def _zrow(i):
        @pl.loop(0, 4)
        def _zcol(j):
            astage[i, pl.ds(j * 16, 16)] = zvec

    @pl.loop(0, NROW // CB)
    def _zcp(t):
        pltpu.sync_copy(astage, zsh.at[pl.ds(s * NROW + t * CB, CB)])
        pltpu.sync_copy(astage, ssh.at[pl.ds(s * NROW + t * CB, CB)])

    plsc.subcore_barrier()

    coff = c * N
    base0 = s * EPT

    @pl.loop(0, NCH)
    def _chunk(i):
        base = base0 + i * CB
        pltpu.sync_copy(src_hbm.at[pl.ds(base, CB)], srcv)
        pltpu.sync_copy(dst_hbm.at[pl.ds(base, CB)], dstv)

        @pl.loop(0, CB // 16)
        def _off(j):
            srcoff[pl.ds(j * 16, 16)] = srcv[pl.ds(j * 16, 16)] + coff
            dstoff[pl.ds(j * 16, 16)] = dstv[pl.ds(j * 16, 16)] + coff

        pltpu.sync_copy(kf.at[srcoff], kg)
        pltpu.sync_copy(qf.at[dstoff], qg)
        pltpu.sync_copy(vf.at[srcoff], vg)
        pltpu.sync_copy(eef.at[pl.ds(c * NE + base, CB)], eev)

        @pl.loop(0, CB)
        def _edge(j):
            p0 = (kg[j, pl.ds(0, 16)] * qg[j, pl.ds(0, 16)]
                  + kg[j, pl.ds(16, 16)] * qg[j, pl.ds(16, 16)])
            s0 = jnp.sum(p0) * INV_SQRT_DK
            p1 = (kg[j, pl.ds(32, 16)] * qg[j, pl.ds(32, 16)]
                  + kg[j, pl.ds(48, 16)] * qg[j, pl.ds(48, 16)])
            s1 = jnp.sum(p1) * INV_SQRT_DK
            for t in range(4):
                sc = s0 if t < 2 else s1
                a = jnp.exp(sc * (1.0 + eev[j, pl.ds(t * 16, 16)]))
                astage[j, pl.ds(t * 16, 16)] = a
                mstage[j, pl.ds(t * 16, 16)] = vg[j, pl.ds(t * 16, 16)] * a

        pltpu.sync_copy(astage, zsh.at[dstv], add=True)
        pltpu.sync_copy(mstage, ssh.at[dstv], add=True)

    plsc.subcore_barrier()
    r0 = s * NROW
    pltpu.sync_copy(zsh.at[pl.ds(r0, NROW)], z_hbm.at[c, pl.ds(r0, NROW)])
    pltpu.sync_copy(ssh.at[pl.ds(r0, NROW)], s_hbm.at[c, pl.ds(r0, NROW)])


def _edge_stage_sc(K2, Q2, V2, Ee2, src, dst):
    kf = K2.reshape(2 * N, 64)
    qf = Q2.reshape(2 * N, 64)
    vf = V2.reshape(2 * N, 64)
    eef = Ee2.reshape(2 * NE, 64)
    mesh = plsc.VectorSubcoreMesh(core_axis_name="c", subcore_axis_name="s")
    cp = pltpu.CompilerParams()
    if "needs_layout_passes" in pltpu.CompilerParams.__dataclass_fields__:
        cp = dataclasses.replace(cp, needs_layout_passes=False)
    if "use_tc_tiling_on_sc" in pltpu.CompilerParams.__dataclass_fields__:
        cp = dataclasses.replace(cp, use_tc_tiling_on_sc=False)
    run = pl.kernel(
        _edge_body,
        compiler_params=cp,
        out_type=[jax.ShapeDtypeStruct((2, NP, 64), jnp.float32),
                  jax.ShapeDtypeStruct((2, NP, 64), jnp.float32)],
        mesh=mesh,
        scratch_types=[
            pltpu.VMEM((CB,), jnp.int32),
            pltpu.VMEM((CB,), jnp.int32),
            pltpu.VMEM((CB,), jnp.int32),
            pltpu.VMEM((CB,), jnp.int32),
            pltpu.VMEM((CB, 64), jnp.float32),
            pltpu.VMEM((CB, 64), jnp.float32),
            pltpu.VMEM((CB, 64), jnp.float32),
            pltpu.VMEM((CB, 64), jnp.float32),
            pltpu.VMEM((CB, 64), jnp.float32),
            pltpu.VMEM((CB, 64), jnp.float32),
            pltpu.VMEM_SHARED((NP, 64), jnp.float32),
            pltpu.VMEM_SHARED((NP, 64), jnp.float32),
        ],
    )
    return run(kf, qf, vf, eef, src, dst)


def _layer_norm(x, g, b):
    mu = jnp.mean(x, axis=-1, keepdims=True)
    xc = x - mu
    var = jnp.mean(xc * xc, axis=-1, keepdims=True)
    return xc * jax.lax.rsqrt(var + 1e-5) * g + b


def _epi_body(h_ref, z_ref, s_ref, wlin_ref, blin_ref, g1_ref, b1_ref,
              wf1_ref, bf1_ref, wf2_ref, bf2_ref, g2_ref, b2_ref, o_ref):
    z = jnp.concatenate([z_ref[0], z_ref[1]], axis=-1)
    s = jnp.concatenate([s_ref[0], s_ref[1]], axis=-1)
    h_out = s / jnp.maximum(z, 1e-6)
    hb = h_ref[...]
    hr = hb + jnp.dot(h_out, wlin_ref[...], preferred_element_type=jnp.float32) + blin_ref[...]
    hr = _layer_norm(hr, g1_ref[...], b1_ref[...])
    t = jnp.dot(hr, wf1_ref[...], preferred_element_type=jnp.float32) + bf1_ref[...]
    t = jnp.maximum(t, 0.0)
    h2 = jnp.dot(t, wf2_ref[...], preferred_element_type=jnp.float32) + bf2_ref[...]
    o_ref[...] = _layer_norm(hr + h2, g2_ref[...], b2_ref[...])


def _epilogue(h, Z2, S2, Wlin, blin, g1, b1, Wf1, bf1, Wf2, bf2, g2, b2):
    full = lambda r, c: pl.BlockSpec((r, c), lambda i: (0, 0))
    vec = lambda c: pl.BlockSpec((c,), lambda i: (0,))
    return pl.pallas_call(
        _epi_body,
        grid=(N // BN,),
        in_specs=[
            pl.BlockSpec((BN, D), lambda i: (i, 0)),
            pl.BlockSpec((2, BN, 64), lambda i: (0, i, 0)),
            pl.BlockSpec((2, BN, 64), lambda i: (0, i, 0)),
            full(D, D), vec(D), vec(D), vec(D),
            full(D, 2 * D), vec(2 * D), full(2 * D, D), vec(D),
            vec(D), vec(D),
        ],
        out_specs=pl.BlockSpec((BN, D), lambda i: (i, 0)),
        out_shape=jax.ShapeDtypeStruct((N, D), jnp.float32),
    )(h, Z2, S2, Wlin, blin, g1, b1, Wf1, bf1, Wf2, bf2, g2, b2)


def kernel(h, e, edge_index, Wq, Wk, Wv, We, Wlin, blin, g1, b1,
           Wf1, bf1, Wf2, bf2, g2, b2):
    src = edge_index[0]
    dst = edge_index[1]
    Q2, K2, V2 = _qkv(h, Wq, Wk, Wv)
    Ee2 = _ee(e, We)
    Z2, S2 = _edge_stage_sc(K2, Q2, V2, Ee2, src, dst)
    return _epilogue(h, Z2, S2, Wlin, blin, g1, b1, Wf1, bf1, Wf2, bf2, g2, b2)


# speedup vs baseline: 38.3258x; 1.0241x over previous
import dataclasses
import functools
import math

import jax
import jax.numpy as jnp
from jax import lax
from jax.experimental import pallas as pl
from jax.experimental.pallas import tpu as pltpu
from jax.experimental.pallas import tpu_sc as plsc

N = 10000
NE = 320000
D = 128
H = 4
DK = D // H
INV_SQRT_DK = 1.0 / math.sqrt(DK)

BN = 1000
BE = 2000

NSC = 2
NSUB = 16
EPT = NE // NSUB
CB = 80
NCH = EPT // CB
NP = 10240
NROW = NP // NSUB


def _qkv_body(h_ref, wq_ref, wk_ref, wv_ref, q_ref, kv_ref):
    hb = h_ref[...]
    qb = jnp.dot(hb, wq_ref[...], preferred_element_type=jnp.float32)
    kb = jnp.dot(hb, wk_ref[...], preferred_element_type=jnp.float32)
    vb = jnp.dot(hb, wv_ref[...], preferred_element_type=jnp.float32)
    q_ref[0] = qb[:, :64]
    q_ref[1] = qb[:, 64:]
    kv_ref[0] = jnp.concatenate([kb[:, :64], vb[:, :64]], axis=1)
    kv_ref[1] = jnp.concatenate([kb[:, 64:], vb[:, 64:]], axis=1)


def _qkv(h, Wq, Wk, Wv):
    wspec = lambda: pl.BlockSpec((D, D), lambda i: (0, 0))
    return pl.pallas_call(
        _qkv_body,
        grid=(N // BN,),
        in_specs=[pl.BlockSpec((BN, D), lambda i: (i, 0)),
                  wspec(), wspec(), wspec()],
        out_specs=[pl.BlockSpec((2, BN, 64), lambda i: (0, i, 0)),
                   pl.BlockSpec((2, BN, 128), lambda i: (0, i, 0))],
        out_shape=[jax.ShapeDtypeStruct((2, N, 64), jnp.float32),
                   jax.ShapeDtypeStruct((2, N, 128), jnp.float32)],
    )(h, Wq, Wk, Wv)


def _ee_body(e_ref, we_ref, o_ref):
    eb = jnp.dot(e_ref[...], we_ref[...], preferred_element_type=jnp.float32)
    o_ref[0] = eb[:, :64]
    o_ref[1] = eb[:, 64:]


def _ee(e, We):
    return pl.pallas_call(
        _ee_body,
        grid=(NE // BE,),
        in_specs=[pl.BlockSpec((BE, D), lambda i: (i, 0)),
                  pl.BlockSpec((D, D), lambda i: (0, 0))],
        out_specs=pl.BlockSpec((2, BE, 64), lambda i: (0, i, 0)),
        out_shape=jax.ShapeDtypeStruct((2, NE, 64), jnp.float32),
    )(e, We)


def _edge_body(kvt, qt, eet, src_hbm, dst_hbm, z_hbm, s_hbm,
               srcv0, srcv1, dstv0, dstv1, dstoff0, dstoff1, dsts,
               kvg0, kvg1, qg0, qg1, eev, astage, mstage, zsh, ssh,
               skv0, skv1, sq0, sq1, ssi0, ssi1, sdi0, sdi1, ssa, ssm):
    c = lax.axis_index("c")
    s_ = lax.axis_index("s")
    coff = c * N
    base0 = s_ * EPT

    srcv = (srcv0, srcv1)
    dstv = (dstv0, dstv1)
    dstoff = (dstoff0, dstoff1)
    kvg = (kvg0, kvg1)
    qg = (qg0, qg1)
    skv = (skv0, skv1)
    sq = (sq0, sq1)
    ssi = (ssi0, ssi1)
    sdi = (sdi0, sdi1)

    def offs(sl):
        @pl.loop(0, CB // 16)
        def _o(j):
            srcv[sl][pl.ds(j * 16, 16)] = srcv[sl][pl.ds(j * 16, 16)] + coff
            dstoff[sl][pl.ds(j * 16, 16)] = dstv[sl][pl.ds(j * 16, 16)] + coff

    def issue_gathers(sl):
        pltpu.async_copy(kvt.at[srcv[sl]], kvg[sl], skv[sl])
        pltpu.async_copy(qt.at[dstoff[sl]], qg[sl], sq[sl])

    def wait_gathers(sl):
        pltpu.make_async_copy(kvt.at[srcv[sl]], kvg[sl], skv[sl]).wait()
        pltpu.make_async_copy(qt.at[dstoff[sl]], qg[sl], sq[sl]).wait()

    def issue_idx(sl, base):
        pltpu.async_copy(src_hbm.at[pl.ds(base, CB)], srcv[sl], ssi[sl])
        pltpu.async_copy(dst_hbm.at[pl.ds(base, CB)], dstv[sl], sdi[sl])

    def wait_idx(sl, base):
        pltpu.make_async_copy(src_hbm.at[pl.ds(base, CB)], srcv[sl], ssi[sl]).wait()
        pltpu.make_async_copy(dst_hbm.at[pl.ds(base, CB)], dstv[sl], sdi[sl]).wait()

    pltpu.sync_copy(src_hbm.at[pl.ds(base0, CB)], srcv0)
    pltpu.sync_copy(dst_hbm.at[pl.ds(base0, CB)], dstv0)
    offs(0)
    issue_gathers(0)
    issue_idx(1, base0 + CB)

    zvec = jnp.zeros((16,), jnp.float32)

    @pl.loop(0, CB)
    def _zrow(i):
        @pl.loop(0, 4)
        def _zcol(j):
            astage[i, pl.ds(j * 16, 16)] = zvec

    @pl.loop(0, NROW // CB)
    def _zcp(t):
        pltpu.sync_copy(astage, zsh.at[pl.ds(s_ * NROW + t * CB, CB)])
        pltpu.sync_copy(astage, ssh.at[pl.ds(s_ * NROW + t * CB, CB)])

    plsc.subcore_barrier()

    @pl.loop(0, NCH, step=2)
    def _pair(ii):
        for sl in (0, 1):
            osl = 1 - sl
            chunk = ii + sl
            base = base0 + chunk * CB

            @pl.when(chunk + 1 < NCH)
            def _a():
                wait_idx(osl, base + CB)
                offs(osl)
                issue_gathers(osl)

            wait_gathers(sl)
            pltpu.sync_copy(eet.at[pl.ds(c * NE + base, CB)], eev)

            @pl.when(chunk > 0)
            def _e():
                pltpu.make_async_copy(astage, zsh.at[dsts], ssa).wait()
                pltpu.make_async_copy(mstage, ssh.at[dsts], ssm).wait()

            @pl.loop(0, CB // 16)
            def _cp(j):
                dsts[pl.ds(j * 16, 16)] = dstv[sl][pl.ds(j * 16, 16)]

            @pl.when(chunk + 2 < NCH)
            def _g():
                issue_idx(sl, base + 2 * CB)

            @pl.loop(0, CB)
            def _edge(j):
                p0 = (kvg[sl][j, pl.ds(0, 16)] * qg[sl][j, pl.ds(0, 16)]
                      + kvg[sl][j, pl.ds(16, 16)] * qg[sl][j, pl.ds(16, 16)])
                s0 = jnp.sum(p0) * INV_SQRT_DK
                p1 = (kvg[sl][j, pl.ds(32, 16)] * qg[sl][j, pl.ds(32, 16)]
                      + kvg[sl][j, pl.ds(48, 16)] * qg[sl][j, pl.ds(48, 16)])
                s1 = jnp.sum(p1) * INV_SQRT_DK
                for t in range(4):
                    sc_ = s0 if t < 2 else s1
                    a = jnp.exp(sc_ * (1.0 + eev[j, pl.ds(t * 16, 16)]))
                    astage[j, pl.ds(t * 16, 16)] = a
                    mstage[j, pl.ds(t * 16, 16)] = (
                        kvg[sl][j, pl.ds(64 + t * 16, 16)] * a)

            pltpu.async_copy(astage, zsh.at[dsts], ssa, add=True)
            pltpu.async_copy(mstage, ssh.at[dsts], ssm, add=True)

    pltpu.make_async_copy(astage, zsh.at[dsts], ssa).wait()
    pltpu.make_async_copy(mstage, ssh.at[dsts], ssm).wait()
    plsc.subcore_barrier()

    r0 = s_ * NROW
    pltpu.sync_copy(zsh.at[pl.ds(r0, NROW)], z_hbm.at[c, pl.ds(r0, NROW)])
    pltpu.sync_copy(ssh.at[pl.ds(r0, NROW)], s_hbm.at[c, pl.ds(r0, NROW)])


def _edge_stage_sc(KV2, Q2, Ee2, src, dst):
    kvt = KV2.reshape(2 * N, 128)
    qt = Q2.reshape(2 * N, 64)
    eet = Ee2.reshape(2 * NE, 64)
    mesh = plsc.VectorSubcoreMesh(core_axis_name="c", subcore_axis_name="s")
    cp = pltpu.CompilerParams()
    if "needs_layout_passes" in pltpu.CompilerParams.__dataclass_fields__:
        cp = dataclasses.replace(cp, needs_layout_passes=False)
    if "use_tc_tiling_on_sc" in pltpu.CompilerParams.__dataclass_fields__:
        cp = dataclasses.replace(cp, use_tc_tiling_on_sc=False)
    run = pl.kernel(
        _edge_body,
        compiler_params=cp,
        out_type=[jax.ShapeDtypeStruct((2, NP, 64), jnp.float32),
                  jax.ShapeDtypeStruct((2, NP, 64), jnp.float32)],
        mesh=mesh,
        scratch_types=[
            pltpu.VMEM((CB,), jnp.int32),
            pltpu.VMEM((CB,), jnp.int32),
            pltpu.VMEM((CB,), jnp.int32),
            pltpu.VMEM((CB,), jnp.int32),
            pltpu.VMEM((CB,), jnp.int32),
            pltpu.VMEM((CB,), jnp.int32),
            pltpu.VMEM((CB,), jnp.int32),
            pltpu.VMEM((CB, 128), jnp.float32),
            pltpu.VMEM((CB, 128), jnp.float32),
            pltpu.VMEM((CB, 64), jnp.float32),
            pltpu.VMEM((CB, 64), jnp.float32),
            pltpu.VMEM((CB, 64), jnp.float32),
            pltpu.VMEM((CB, 64), jnp.float32),
            pltpu.VMEM((CB, 64), jnp.float32),
            pltpu.VMEM_SHARED((NP, 64), jnp.float32),
            pltpu.VMEM_SHARED((NP, 64), jnp.float32),
            pltpu.SemaphoreType.DMA,
            pltpu.SemaphoreType.DMA,
            pltpu.SemaphoreType.DMA,
            pltpu.SemaphoreType.DMA,
            pltpu.SemaphoreType.DMA,
            pltpu.SemaphoreType.DMA,
            pltpu.SemaphoreType.DMA,
            pltpu.SemaphoreType.DMA,
            pltpu.SemaphoreType.DMA,
            pltpu.SemaphoreType.DMA,
        ],
    )
    return run(kvt, qt, eet, src, dst)


def _layer_norm(x, g, b):
    mu = jnp.mean(x, axis=-1, keepdims=True)
    xc = x - mu
    var = jnp.mean(xc * xc, axis=-1, keepdims=True)
    return xc * jax.lax.rsqrt(var + 1e-5) * g + b


def _epi_body(h_ref, z_ref, s_ref, wlin_ref, blin_ref, g1_ref, b1_ref,
              wf1_ref, bf1_ref, wf2_ref, bf2_ref, g2_ref, b2_ref, o_ref):
    z = jnp.concatenate([z_ref[0], z_ref[1]], axis=-1)
    s = jnp.concatenate([s_ref[0], s_ref[1]], axis=-1)
    h_out = s / jnp.maximum(z, 1e-6)
    hb = h_ref[...]
    hr = hb + jnp.dot(h_out, wlin_ref[...], preferred_element_type=jnp.float32) + blin_ref[...]
    hr = _layer_norm(hr, g1_ref[...], b1_ref[...])
    t = jnp.dot(hr, wf1_ref[...], preferred_element_type=jnp.float32) + bf1_ref[...]
    t = jnp.maximum(t, 0.0)
    h2 = jnp.dot(t, wf2_ref[...], preferred_element_type=jnp.float32) + bf2_ref[...]
    o_ref[...] = _layer_norm(hr + h2, g2_ref[...], b2_ref[...])


def _epilogue(h, Z2, S2, Wlin, blin, g1, b1, Wf1, bf1, Wf2, bf2, g2, b2):
    full = lambda r, c: pl.BlockSpec((r, c), lambda i: (0, 0))
    vec = lambda c: pl.BlockSpec((c,), lambda i: (0,))
    return pl.pallas_call(
        _epi_body,
        grid=(N // BN,),
        in_specs=[
            pl.BlockSpec((BN, D), lambda i: (i, 0)),
            pl.BlockSpec((2, BN, 64), lambda i: (0, i, 0)),
            pl.BlockSpec((2, BN, 64), lambda i: (0, i, 0)),
            full(D, D), vec(D), vec(D), vec(D),
            full(D, 2 * D), vec(2 * D), full(2 * D, D), vec(D),
            vec(D), vec(D),
        ],
        out_specs=pl.BlockSpec((BN, D), lambda i: (i, 0)),
        out_shape=jax.ShapeDtypeStruct((N, D), jnp.float32),
    )(h, Z2, S2, Wlin, blin, g1, b1, Wf1, bf1, Wf2, bf2, g2, b2)


def kernel(h, e, edge_index, Wq, Wk, Wv, We, Wlin, blin, g1, b1,
           Wf1, bf1, Wf2, bf2, g2, b2):
    src = edge_index[0]
    dst = edge_index[1]
    Q2, KV2 = _qkv(h, Wq, Wk, Wv)
    Ee2 = _ee(e, We)
    Z2, S2 = _edge_stage_sc(KV2, Q2, Ee2, src, dst)
    return _epilogue(h, Z2, S2, Wlin, blin, g1, b1, Wf1, bf1, Wf2, bf2, g2, b2)


# speedup vs baseline: 64.6034x; 1.6856x over previous
import dataclasses
import functools
import math

import jax
import jax.numpy as jnp
from jax import lax
from jax.experimental import pallas as pl
from jax.experimental.pallas import tpu as pltpu
from jax.experimental.pallas import tpu_sc as plsc

N = 10000
NE = 320000
D = 128
H = 4
DK = D // H
INV_SQRT_DK = 1.0 / math.sqrt(DK)

BN = 1000
BE = 2000

NSC = 2
NSUB = 16
EPT = NE // NSUB
CB = 80
NCH = EPT // CB
NP = 10240
NROW = NP // NSUB


def _qkv_body(h_ref, wq_ref, wk_ref, wv_ref, q_ref, kv_ref):
    hb = h_ref[...]
    qb = jnp.dot(hb, wq_ref[...], preferred_element_type=jnp.float32)
    kb = jnp.dot(hb, wk_ref[...], preferred_element_type=jnp.float32)
    vb = jnp.dot(hb, wv_ref[...], preferred_element_type=jnp.float32)
    q_ref[0] = qb[:, :64]
    q_ref[1] = qb[:, 64:]
    kv_ref[0] = jnp.concatenate([kb[:, :64], vb[:, :64]], axis=1)
    kv_ref[1] = jnp.concatenate([kb[:, 64:], vb[:, 64:]], axis=1)


def _qkv(h, Wq, Wk, Wv):
    wspec = lambda: pl.BlockSpec((D, D), lambda i: (0, 0))
    return pl.pallas_call(
        _qkv_body,
        grid=(N // BN,),
        in_specs=[pl.BlockSpec((BN, D), lambda i: (i, 0)),
                  wspec(), wspec(), wspec()],
        out_specs=[pl.BlockSpec((2, BN, 64), lambda i: (0, i, 0)),
                   pl.BlockSpec((2, BN, 128), lambda i: (0, i, 0))],
        out_shape=[jax.ShapeDtypeStruct((2, N, 64), jnp.float32),
                   jax.ShapeDtypeStruct((2, N, 128), jnp.float32)],
    )(h, Wq, Wk, Wv)


def _ee_body(e_ref, we_ref, o_ref):
    eb = jnp.dot(e_ref[...], we_ref[...], preferred_element_type=jnp.float32)
    o_ref[0] = eb[:, :64]
    o_ref[1] = eb[:, 64:]


def _ee(e, We):
    return pl.pallas_call(
        _ee_body,
        grid=(NE // BE,),
        in_specs=[pl.BlockSpec((BE, D), lambda i: (i, 0)),
                  pl.BlockSpec((D, D), lambda i: (0, 0))],
        out_specs=pl.BlockSpec((2, BE, 64), lambda i: (0, i, 0)),
        out_shape=jax.ShapeDtypeStruct((2, NE, 64), jnp.float32),
    )(e, We)


def _edge_body(kvt, qt, eet, src_hbm, dst_hbm, z_hbm, s_hbm,
               srcv0, srcv1, dstv0, dstv1, dstoff0, dstoff1, dsts,
               kvg0, kvg1, qg0, qg1, eev, astage, mstage, zsh, ssh,
               skv0, skv1, sq0, sq1, ssi0, ssi1, sdi0, sdi1, ssa, ssm):
    c = lax.axis_index("c")
    s_ = lax.axis_index("s")
    coff = c * N
    base0 = s_ * EPT

    srcv = (srcv0, srcv1)
    dstv = (dstv0, dstv1)
    dstoff = (dstoff0, dstoff1)
    kvg = (kvg0, kvg1)
    qg = (qg0, qg1)
    skv = (skv0, skv1)
    sq = (sq0, sq1)
    ssi = (ssi0, ssi1)
    sdi = (sdi0, sdi1)

    def offs(sl):
        @pl.loop(0, CB // 16)
        def _o(j):
            srcv[sl][pl.ds(j * 16, 16)] = srcv[sl][pl.ds(j * 16, 16)] + coff
            dstoff[sl][pl.ds(j * 16, 16)] = dstv[sl][pl.ds(j * 16, 16)] + coff

    def issue_gathers(sl):
        pltpu.async_copy(kvt.at[srcv[sl]], kvg[sl], skv[sl])
        pltpu.async_copy(qt.at[dstoff[sl]], qg[sl], sq[sl])

    def wait_gathers(sl):
        pltpu.make_async_copy(kvt.at[srcv[sl]], kvg[sl], skv[sl]).wait()
        pltpu.make_async_copy(qt.at[dstoff[sl]], qg[sl], sq[sl]).wait()

    def issue_idx(sl, base):
        pltpu.async_copy(src_hbm.at[pl.ds(base, CB)], srcv[sl], ssi[sl])
        pltpu.async_copy(dst_hbm.at[pl.ds(base, CB)], dstv[sl], sdi[sl])

    def wait_idx(sl, base):
        pltpu.make_async_copy(src_hbm.at[pl.ds(base, CB)], srcv[sl], ssi[sl]).wait()
        pltpu.make_async_copy(dst_hbm.at[pl.ds(base, CB)], dstv[sl], sdi[sl]).wait()

    pltpu.sync_copy(src_hbm.at[pl.ds(base0, CB)], srcv0)
    pltpu.sync_copy(dst_hbm.at[pl.ds(base0, CB)], dstv0)
    offs(0)
    issue_gathers(0)
    issue_idx(1, base0 + CB)

    zvec = jnp.zeros((16,), jnp.float32)

    @pl.loop(0, CB)
    def _zrow(i):
        @pl.loop(0, 4)
        def _zcol(j):
            astage[i, pl.ds(j * 16, 16)] = zvec

    @pl.loop(0, NROW // CB)
    def _zcp(t):
        pltpu.sync_copy(astage, zsh.at[pl.ds(s_ * NROW + t * CB, CB)])
        pltpu.sync_copy(astage, ssh.at[pl.ds(s_ * NROW + t * CB, CB)])

    plsc.subcore_barrier()

    @pl.loop(0, NCH, step=2)
    def _pair(ii):
        for sl in (0, 1):
            osl = 1 - sl
            chunk = ii + sl
            base = base0 + chunk * CB

            @pl.when(chunk + 1 < NCH)
            def _a():
                wait_idx(osl, base + CB)
                offs(osl)
                issue_gathers(osl)

            wait_gathers(sl)
            pltpu.sync_copy(eet.at[pl.ds(c * NE + base, CB)], eev)

            @pl.when(chunk > 0)
            def _e():
                pltpu.make_async_copy(astage, zsh.at[dsts], ssa).wait()
                pltpu.make_async_copy(mstage, ssh.at[dsts], ssm).wait()

            @pl.loop(0, CB // 16)
            def _cp(j):
                dsts[pl.ds(j * 16, 16)] = dstv[sl][pl.ds(j * 16, 16)]

            @pl.when(chunk + 2 < NCH)
            def _g():
                issue_idx(sl, base + 2 * CB)

            @plsc.parallel_loop(0, CB, unroll=4)
            def _edge(j):
                p0 = (kvg[sl][j, pl.ds(0, 16)] * qg[sl][j, pl.ds(0, 16)]
                      + kvg[sl][j, pl.ds(16, 16)] * qg[sl][j, pl.ds(16, 16)])
                s0 = jnp.sum(p0) * INV_SQRT_DK
                p1 = (kvg[sl][j, pl.ds(32, 16)] * qg[sl][j, pl.ds(32, 16)]
                      + kvg[sl][j, pl.ds(48, 16)] * qg[sl][j, pl.ds(48, 16)])
                s1 = jnp.sum(p1) * INV_SQRT_DK
                for t in range(4):
                    sc_ = s0 if t < 2 else s1
                    a = jnp.exp(sc_ * (1.0 + eev[j, pl.ds(t * 16, 16)]))
                    astage[j, pl.ds(t * 16, 16)] = a
                    mstage[j, pl.ds(t * 16, 16)] = (
                        kvg[sl][j, pl.ds(64 + t * 16, 16)] * a)

            pltpu.async_copy(astage, zsh.at[dsts], ssa, add=True)
            pltpu.async_copy(mstage, ssh.at[dsts], ssm, add=True)

    pltpu.make_async_copy(astage, zsh.at[dsts], ssa).wait()
    pltpu.make_async_copy(mstage, ssh.at[dsts], ssm).wait()
    plsc.subcore_barrier()

    r0 = s_ * NROW
    pltpu.sync_copy(zsh.at[pl.ds(r0, NROW)], z_hbm.at[c, pl.ds(r0, NROW)])
    pltpu.sync_copy(ssh.at[pl.ds(r0, NROW)], s_hbm.at[c, pl.ds(r0, NROW)])


def _edge_stage_sc(KV2, Q2, Ee2, src, dst):
    kvt = KV2.reshape(2 * N, 128)
    qt = Q2.reshape(2 * N, 64)
    eet = Ee2.reshape(2 * NE, 64)
    mesh = plsc.VectorSubcoreMesh(core_axis_name="c", subcore_axis_name="s")
    cp = pltpu.CompilerParams()
    if "needs_layout_passes" in pltpu.CompilerParams.__dataclass_fields__:
        cp = dataclasses.replace(cp, needs_layout_passes=False)
    if "use_tc_tiling_on_sc" in pltpu.CompilerParams.__dataclass_fields__:
        cp = dataclasses.replace(cp, use_tc_tiling_on_sc=False)
    run = pl.kernel(
        _edge_body,
        compiler_params=cp,
        out_type=[jax.ShapeDtypeStruct((2, NP, 64), jnp.float32),
                  jax.ShapeDtypeStruct((2, NP, 64), jnp.float32)],
        mesh=mesh,
        scratch_types=[
            pltpu.VMEM((CB,), jnp.int32),
            pltpu.VMEM((CB,), jnp.int32),
            pltpu.VMEM((CB,), jnp.int32),
            pltpu.VMEM((CB,), jnp.int32),
            pltpu.VMEM((CB,), jnp.int32),
            pltpu.VMEM((CB,), jnp.int32),
            pltpu.VMEM((CB,), jnp.int32),
            pltpu.VMEM((CB, 128), jnp.float32),
            pltpu.VMEM((CB, 128), jnp.float32),
            pltpu.VMEM((CB, 64), jnp.float32),
            pltpu.VMEM((CB, 64), jnp.float32),
            pltpu.VMEM((CB, 64), jnp.float32),
            pltpu.VMEM((CB, 64), jnp.float32),
            pltpu.VMEM((CB, 64), jnp.float32),
            pltpu.VMEM_SHARED((NP, 64), jnp.float32),
            pltpu.VMEM_SHARED((NP, 64), jnp.float32),
            pltpu.SemaphoreType.DMA,
            pltpu.SemaphoreType.DMA,
            pltpu.SemaphoreType.DMA,
            pltpu.SemaphoreType.DMA,
            pltpu.SemaphoreType.DMA,
            pltpu.SemaphoreType.DMA,
            pltpu.SemaphoreType.DMA,
            pltpu.SemaphoreType.DMA,
            pltpu.SemaphoreType.DMA,
            pltpu.SemaphoreType.DMA,
        ],
    )
    return run(kvt, qt, eet, src, dst)


def _layer_norm(x, g, b):
    mu = jnp.mean(x, axis=-1, keepdims=True)
    xc = x - mu
    var = jnp.mean(xc * xc, axis=-1, keepdims=True)
    return xc * jax.lax.rsqrt(var + 1e-5) * g + b


def _epi_body(h_ref, z_ref, s_ref, wlin_ref, blin_ref, g1_ref, b1_ref,
              wf1_ref, bf1_ref, wf2_ref, bf2_ref, g2_ref, b2_ref, o_ref):
    z = jnp.concatenate([z_ref[0], z_ref[1]], axis=-1)
    s = jnp.concatenate([s_ref[0], s_ref[1]], axis=-1)
    h_out = s / jnp.maximum(z, 1e-6)
    hb = h_ref[...]
    hr = hb + jnp.dot(h_out, wlin_ref[...], preferred_element_type=jnp.float32) + blin_ref[...]
    hr = _layer_norm(hr, g1_ref[...], b1_ref[...])
    t = jnp.dot(hr, wf1_ref[...], preferred_element_type=jnp.float32) + bf1_ref[...]
    t = jnp.maximum(t, 0.0)
    h2 = jnp.dot(t, wf2_ref[...], preferred_element_type=jnp.float32) + bf2_ref[...]
    o_ref[...] = _layer_norm(hr + h2, g2_ref[...], b2_ref[...])


def _epilogue(h, Z2, S2, Wlin, blin, g1, b1, Wf1, bf1, Wf2, bf2, g2, b2):
    full = lambda r, c: pl.BlockSpec((r, c), lambda i: (0, 0))
    vec = lambda c: pl.BlockSpec((c,), lambda i: (0,))
    return pl.pallas_call(
        _epi_body,
        grid=(N // BN,),
        in_specs=[
            pl.BlockSpec((BN, D), lambda i: (i, 0)),
            pl.BlockSpec((2, BN, 64), lambda i: (0, i, 0)),
            pl.BlockSpec((2, BN, 64), lambda i: (0, i, 0)),
            full(D, D), vec(D), vec(D), vec(D),
            full(D, 2 * D), vec(2 * D), full(2 * D, D), vec(D),
            vec(D), vec(D),
        ],
        out_specs=pl.BlockSpec((BN, D), lambda i: (i, 0)),
        out_shape=jax.ShapeDtypeStruct((N, D), jnp.float32),
    )(h, Z2, S2, Wlin, blin, g1, b1, Wf1, bf1, Wf2, bf2, g2, b2)


def kernel(h, e, edge_index, Wq, Wk, Wv, We, Wlin, blin, g1, b1,
           Wf1, bf1, Wf2, bf2, g2, b2):
    src = edge_index[0]
    dst = edge_index[1]
    Q2, KV2 = _qkv(h, Wq, Wk, Wv)
    Ee2 = _ee(e, We)
    Z2, S2 = _edge_stage_sc(KV2, Q2, Ee2, src, dst)
    return _epilogue(h, Z2, S2, Wlin, blin, g1, b1, Wf1, bf1, Wf2, bf2, g2, b2)


# speedup vs baseline: 64.8138x; 1.0033x over previous
import dataclasses
import functools
import math

import jax
import jax.numpy as jnp
from jax import lax
from jax.experimental import pallas as pl
from jax.experimental.pallas import tpu as pltpu
from jax.experimental.pallas import tpu_sc as plsc

N = 10000
NE = 320000
D = 128
H = 4
DK = D // H
INV_SQRT_DK = 1.0 / math.sqrt(DK)

BN = 1000
BE = 2000

NSC = 2
NSUB = 16
EPT = NE // NSUB
CB = 80
NCH = EPT // CB
NP = 10240
NROW = NP // NSUB


def _qkv_body(h_ref, wq_ref, wk_ref, wv_ref, q_ref, kv_ref):
    hb = h_ref[...]
    qb = jnp.dot(hb, wq_ref[...], preferred_element_type=jnp.float32)
    kb = jnp.dot(hb, wk_ref[...], preferred_element_type=jnp.float32)
    vb = jnp.dot(hb, wv_ref[...], preferred_element_type=jnp.float32)
    q_ref[0] = qb[:, :64]
    q_ref[1] = qb[:, 64:]
    kv_ref[0] = jnp.concatenate([kb[:, :64], vb[:, :64]], axis=1)
    kv_ref[1] = jnp.concatenate([kb[:, 64:], vb[:, 64:]], axis=1)


def _qkv(h, Wq, Wk, Wv):
    wspec = lambda: pl.BlockSpec((D, D), lambda i: (0, 0))
    return pl.pallas_call(
        _qkv_body,
        grid=(N // BN,),
        in_specs=[pl.BlockSpec((BN, D), lambda i: (i, 0)),
                  wspec(), wspec(), wspec()],
        out_specs=[pl.BlockSpec((2, BN, 64), lambda i: (0, i, 0)),
                   pl.BlockSpec((2, BN, 128), lambda i: (0, i, 0))],
        out_shape=[jax.ShapeDtypeStruct((2, N, 64), jnp.float32),
                   jax.ShapeDtypeStruct((2, N, 128), jnp.float32)],
    )(h, Wq, Wk, Wv)


def _ee_body(e_ref, we_ref, o_ref):
    eb = jnp.dot(e_ref[...], we_ref[...], preferred_element_type=jnp.float32)
    o_ref[0] = eb[:, :64]
    o_ref[1] = eb[:, 64:]


def _ee(e, We):
    return pl.pallas_call(
        _ee_body,
        grid=(NE // BE,),
        in_specs=[pl.BlockSpec((BE, D), lambda i: (i, 0)),
                  pl.BlockSpec((D, D), lambda i: (0, 0))],
        out_specs=pl.BlockSpec((2, BE, 64), lambda i: (0, i, 0)),
        out_shape=jax.ShapeDtypeStruct((2, NE, 64), jnp.float32),
    )(e, We)


def _edge_body(kvt, qt, eet, src_hbm, dst_hbm, z_hbm, s_hbm,
               srcv0, srcv1, dstv0, dstv1, dsts,
               kvg0, kvg1, qg0, qg1, eev, astage, mstage, zsh, ssh,
               skv0, skv1, sq0, sq1, ssi0, ssi1, sdi0, sdi1, ssa, ssm):
    c = lax.axis_index("c")
    s_ = lax.axis_index("s")
    base0 = s_ * EPT

    srcv = (srcv0, srcv1)
    dstv = (dstv0, dstv1)
    kvg = (kvg0, kvg1)
    qg = (qg0, qg1)
    skv = (skv0, skv1)
    sq = (sq0, sq1)
    ssi = (ssi0, ssi1)
    sdi = (sdi0, sdi1)

    def issue_gathers(sl):
        pltpu.async_copy(kvt.at[c].at[srcv[sl]], kvg[sl], skv[sl])
        pltpu.async_copy(qt.at[c].at[dstv[sl]], qg[sl], sq[sl])

    def wait_gathers(sl):
        pltpu.make_async_copy(kvt.at[c].at[srcv[sl]], kvg[sl], skv[sl]).wait()
        pltpu.make_async_copy(qt.at[c].at[dstv[sl]], qg[sl], sq[sl]).wait()

    def issue_idx(sl, base):
        pltpu.async_copy(src_hbm.at[pl.ds(base, CB)], srcv[sl], ssi[sl])
        pltpu.async_copy(dst_hbm.at[pl.ds(base, CB)], dstv[sl], sdi[sl])

    def wait_idx(sl, base):
        pltpu.make_async_copy(src_hbm.at[pl.ds(base, CB)], srcv[sl], ssi[sl]).wait()
        pltpu.make_async_copy(dst_hbm.at[pl.ds(base, CB)], dstv[sl], sdi[sl]).wait()

    pltpu.sync_copy(src_hbm.at[pl.ds(base0, CB)], srcv0)
    pltpu.sync_copy(dst_hbm.at[pl.ds(base0, CB)], dstv0)
    issue_gathers(0)
    issue_idx(1, base0 + CB)

    zvec = jnp.zeros((16,), jnp.float32)

    @pl.loop(0, CB)
    def _zrow(i):
        @pl.loop(0, 4)
        def _zcol(j):
            astage[i, pl.ds(j * 16, 16)] = zvec

    @pl.loop(0, NROW // CB)
    def _zcp(t):
        pltpu.sync_copy(astage, zsh.at[pl.ds(s_ * NROW + t * CB, CB)])
        pltpu.sync_copy(astage, ssh.at[pl.ds(s_ * NROW + t * CB, CB)])

    plsc.subcore_barrier()

    @pl.loop(0, NCH, step=2)
    def _pair(ii):
        for sl in (0, 1):
            osl = 1 - sl
            chunk = ii + sl
            base = base0 + chunk * CB

            @pl.when(chunk + 1 < NCH)
            def _a():
                wait_idx(osl, base + CB)
                issue_gathers(osl)

            wait_gathers(sl)
            pltpu.sync_copy(eet.at[c, pl.ds(base, CB)], eev)

            @pl.when(chunk > 0)
            def _e():
                pltpu.make_async_copy(astage, zsh.at[dsts], ssa).wait()
                pltpu.make_async_copy(mstage, ssh.at[dsts], ssm).wait()

            @pl.loop(0, CB // 16)
            def _cp(j):
                dsts[pl.ds(j * 16, 16)] = dstv[sl][pl.ds(j * 16, 16)]

            @pl.when(chunk + 2 < NCH)
            def _g():
                issue_idx(sl, base + 2 * CB)

            @plsc.parallel_loop(0, CB, unroll=4)
            def _edge(j):
                p0 = (kvg[sl][j, pl.ds(0, 16)] * qg[sl][j, pl.ds(0, 16)]
                      + kvg[sl][j, pl.ds(16, 16)] * qg[sl][j, pl.ds(16, 16)])
                s0 = jnp.sum(p0) * INV_SQRT_DK
                p1 = (kvg[sl][j, pl.ds(32, 16)] * qg[sl][j, pl.ds(32, 16)]
                      + kvg[sl][j, pl.ds(48, 16)] * qg[sl][j, pl.ds(48, 16)])
                s1 = jnp.sum(p1) * INV_SQRT_DK
                for t in range(4):
                    sc_ = s0 if t < 2 else s1
                    a = jnp.exp(sc_ * (1.0 + eev[j, pl.ds(t * 16, 16)]))
                    astage[j, pl.ds(t * 16, 16)] = a
                    mstage[j, pl.ds(t * 16, 16)] = (
                        kvg[sl][j, pl.ds(64 + t * 16, 16)] * a)

            pltpu.async_copy(astage, zsh.at[dsts], ssa, add=True)
            pltpu.async_copy(mstage, ssh.at[dsts], ssm, add=True)

    pltpu.make_async_copy(astage, zsh.at[dsts], ssa).wait()
    pltpu.make_async_copy(mstage, ssh.at[dsts], ssm).wait()
    plsc.subcore_barrier()

    r0 = s_ * NROW
    pltpu.sync_copy(zsh.at[pl.ds(r0, NROW)], z_hbm.at[c, pl.ds(r0, NROW)])
    pltpu.sync_copy(ssh.at[pl.ds(r0, NROW)], s_hbm.at[c, pl.ds(r0, NROW)])


def _edge_stage_sc(KV2, Q2, Ee2, src, dst):
    kvt = KV2
    qt = Q2
    eet = Ee2
    mesh = plsc.VectorSubcoreMesh(core_axis_name="c", subcore_axis_name="s")
    cp = pltpu.CompilerParams()
    if "needs_layout_passes" in pltpu.CompilerParams.__dataclass_fields__:
        cp = dataclasses.replace(cp, needs_layout_passes=False)
    if "use_tc_tiling_on_sc" in pltpu.CompilerParams.__dataclass_fields__:
        cp = dataclasses.replace(cp, use_tc_tiling_on_sc=False)
    run = pl.kernel(
        _edge_body,
        compiler_params=cp,
        out_type=[jax.ShapeDtypeStruct((2, NP, 64), jnp.float32),
                  jax.ShapeDtypeStruct((2, NP, 64), jnp.float32)],
        mesh=mesh,
        scratch_types=[
            pltpu.VMEM((CB,), jnp.int32),
            pltpu.VMEM((CB,), jnp.int32),
            pltpu.VMEM((CB,), jnp.int32),
            pltpu.VMEM((CB,), jnp.int32),
            pltpu.VMEM((CB,), jnp.int32),
            pltpu.VMEM((CB, 128), jnp.float32),
            pltpu.VMEM((CB, 128), jnp.float32),
            pltpu.VMEM((CB, 64), jnp.float32),
            pltpu.VMEM((CB, 64), jnp.float32),
            pltpu.VMEM((CB, 64), jnp.float32),
            pltpu.VMEM((CB, 64), jnp.float32),
            pltpu.VMEM((CB, 64), jnp.float32),
            pltpu.VMEM_SHARED((NP, 64), jnp.float32),
            pltpu.VMEM_SHARED((NP, 64), jnp.float32),
            pltpu.SemaphoreType.DMA,
            pltpu.SemaphoreType.DMA,
            pltpu.SemaphoreType.DMA,
            pltpu.SemaphoreType.DMA,
            pltpu.SemaphoreType.DMA,
            pltpu.SemaphoreType.DMA,
            pltpu.SemaphoreType.DMA,
            pltpu.SemaphoreType.DMA,
            pltpu.SemaphoreType.DMA,
            pltpu.SemaphoreType.DMA,
        ],
    )
    return run(kvt, qt, eet, src, dst)


def _layer_norm(x, g, b):
    mu = jnp.mean(x, axis=-1, keepdims=True)
    xc = x - mu
    var = jnp.mean(xc * xc, axis=-1, keepdims=True)
    return xc * jax.lax.rsqrt(var + 1e-5) * g + b


def _epi_body(h_ref, z_ref, s_ref, wlin_ref, blin_ref, g1_ref, b1_ref,
              wf1_ref, bf1_ref, wf2_ref, bf2_ref, g2_ref, b2_ref, o_ref):
    z = jnp.concatenate([z_ref[0], z_ref[1]], axis=-1)
    s = jnp.concatenate([s_ref[0], s_ref[1]], axis=-1)
    h_out = s / jnp.maximum(z, 1e-6)
    hb = h_ref[...]
    hr = hb + jnp.dot(h_out, wlin_ref[...], preferred_element_type=jnp.float32) + blin_ref[...]
    hr = _layer_norm(hr, g1_ref[...], b1_ref[...])
    t = jnp.dot(hr, wf1_ref[...], preferred_element_type=jnp.float32) + bf1_ref[...]
    t = jnp.maximum(t, 0.0)
    h2 = jnp.dot(t, wf2_ref[...], preferred_element_type=jnp.float32) + bf2_ref[...]
    o_ref[...] = _layer_norm(hr + h2, g2_ref[...], b2_ref[...])


def _epilogue(h, Z2, S2, Wlin, blin, g1, b1, Wf1, bf1, Wf2, bf2, g2, b2):
    full = lambda r, c: pl.BlockSpec((r, c), lambda i: (0, 0))
    vec = lambda c: pl.BlockSpec((c,), lambda i: (0,))
    return pl.pallas_call(
        _epi_body,
        grid=(N // BN,),
        in_specs=[
            pl.BlockSpec((BN, D), lambda i: (i, 0)),
            pl.BlockSpec((2, BN, 64), lambda i: (0, i, 0)),
            pl.BlockSpec((2, BN, 64), lambda i: (0, i, 0)),
            full(D, D), vec(D), vec(D), vec(D),
            full(D, 2 * D), vec(2 * D), full(2 * D, D), vec(D),
            vec(D), vec(D),
        ],
        out_specs=pl.BlockSpec((BN, D), lambda i: (i, 0)),
        out_shape=jax.ShapeDtypeStruct((N, D), jnp.float32),
    )(h, Z2, S2, Wlin, blin, g1, b1, Wf1, bf1, Wf2, bf2, g2, b2)


def kernel(h, e, edge_index, Wq, Wk, Wv, We, Wlin, blin, g1, b1,
           Wf1, bf1, Wf2, bf2, g2, b2):
    src = edge_index[0]
    dst = edge_index[1]
    Q2, KV2 = _qkv(h, Wq, Wk, Wv)
    Ee2 = _ee(e, We)
    Z2, S2 = _edge_stage_sc(KV2, Q2, Ee2, src, dst)
    return _epilogue(h, Z2, S2, Wlin, blin, g1, b1, Wf1, bf1, Wf2, bf2, g2, b2)


# speedup vs baseline: 67.3200x; 1.0387x over previous
import dataclasses
import functools
import math

import jax
import jax.numpy as jnp
from jax import lax
from jax.experimental import pallas as pl
from jax.experimental.pallas import tpu as pltpu
from jax.experimental.pallas import tpu_sc as plsc

N = 10000
NE = 320000
D = 128
H = 4
DK = D // H
INV_SQRT_DK = 1.0 / math.sqrt(DK)

BN = 1000
BE = 2000

NSC = 2
NSUB = 16
EPT = NE // NSUB
CB = 80
NCH = EPT // CB
NP = 10240
NROW = NP // NSUB


def _qkv_body(h_ref, wq_ref, wk_ref, wv_ref, q_ref, kv_ref):
    hb = h_ref[...].astype(jnp.bfloat16)
    qb = jnp.dot(hb, wq_ref[...].astype(jnp.bfloat16),
                 preferred_element_type=jnp.float32)
    kb = jnp.dot(hb, wk_ref[...].astype(jnp.bfloat16),
                 preferred_element_type=jnp.float32)
    vb = jnp.dot(hb, wv_ref[...].astype(jnp.bfloat16),
                 preferred_element_type=jnp.float32)
    q_ref[0] = qb[:, :64]
    q_ref[1] = qb[:, 64:]
    kv_ref[0] = jnp.concatenate([kb[:, :64], vb[:, :64]], axis=1)
    kv_ref[1] = jnp.concatenate([kb[:, 64:], vb[:, 64:]], axis=1)


def _qkv(h, Wq, Wk, Wv):
    wspec = lambda: pl.BlockSpec((D, D), lambda i: (0, 0))
    return pl.pallas_call(
        _qkv_body,
        grid=(N // BN,),
        in_specs=[pl.BlockSpec((BN, D), lambda i: (i, 0)),
                  wspec(), wspec(), wspec()],
        out_specs=[pl.BlockSpec((2, BN, 64), lambda i: (0, i, 0)),
                   pl.BlockSpec((2, BN, 128), lambda i: (0, i, 0))],
        out_shape=[jax.ShapeDtypeStruct((2, N, 64), jnp.float32),
                   jax.ShapeDtypeStruct((2, N, 128), jnp.float32)],
    )(h, Wq, Wk, Wv)


def _ee_body(e_ref, we_ref, o_ref):
    eb = jnp.dot(e_ref[...].astype(jnp.bfloat16),
                 we_ref[...].astype(jnp.bfloat16),
                 preferred_element_type=jnp.float32)
    o_ref[0] = eb[:, :64]
    o_ref[1] = eb[:, 64:]


def _ee(e, We):
    return pl.pallas_call(
        _ee_body,
        grid=(NE // BE,),
        in_specs=[pl.BlockSpec((BE, D), lambda i: (i, 0)),
                  pl.BlockSpec((D, D), lambda i: (0, 0))],
        out_specs=pl.BlockSpec((2, BE, 64), lambda i: (0, i, 0)),
        out_shape=jax.ShapeDtypeStruct((2, NE, 64), jnp.float32),
    )(e, We)


def _edge_body(kvt, qt, eet, src_hbm, dst_hbm, acc_hbm,
               srcv0, srcv1, dstv0, dstv1, dsts,
               kvg0, kvg1, qg0, qg1, eev, stage, acc,
               skv0, skv1, sq0, sq1, ssi0, ssi1, sdi0, sdi1, ssc):
    c = lax.axis_index("c")
    s_ = lax.axis_index("s")
    base0 = s_ * EPT

    srcv = (srcv0, srcv1)
    dstv = (dstv0, dstv1)
    kvg = (kvg0, kvg1)
    qg = (qg0, qg1)
    skv = (skv0, skv1)
    sq = (sq0, sq1)
    ssi = (ssi0, ssi1)
    sdi = (sdi0, sdi1)

    def issue_gathers(sl):
        pltpu.async_copy(kvt.at[c].at[srcv[sl]], kvg[sl], skv[sl])
        pltpu.async_copy(qt.at[c].at[dstv[sl]], qg[sl], sq[sl])

    def wait_gathers(sl):
        pltpu.make_async_copy(kvt.at[c].at[srcv[sl]], kvg[sl], skv[sl]).wait()
        pltpu.make_async_copy(qt.at[c].at[dstv[sl]], qg[sl], sq[sl]).wait()

    def issue_idx(sl, base):
        pltpu.async_copy(src_hbm.at[pl.ds(base, CB)], srcv[sl], ssi[sl])
        pltpu.async_copy(dst_hbm.at[pl.ds(base, CB)], dstv[sl], sdi[sl])

    def wait_idx(sl, base):
        pltpu.make_async_copy(src_hbm.at[pl.ds(base, CB)], srcv[sl], ssi[sl]).wait()
        pltpu.make_async_copy(dst_hbm.at[pl.ds(base, CB)], dstv[sl], sdi[sl]).wait()

    pltpu.sync_copy(src_hbm.at[pl.ds(base0, CB)], srcv0)
    pltpu.sync_copy(dst_hbm.at[pl.ds(base0, CB)], dstv0)
    issue_gathers(0)
    issue_idx(1, base0 + CB)

    zvec = jnp.zeros((16,), jnp.float32)

    @pl.loop(0, CB)
    def _zrow(i):
        @pl.loop(0, 8)
        def _zcol(j):
            stage[i, pl.ds(j * 16, 16)] = zvec

    @pl.loop(0, NROW // CB)
    def _zcp(t):
        pltpu.sync_copy(stage, acc.at[pl.ds(s_ * NROW + t * CB, CB)])

    plsc.subcore_barrier()

    @pl.loop(0, NCH, step=2)
    def _pair(ii):
        for sl in (0, 1):
            osl = 1 - sl
            chunk = ii + sl
            base = base0 + chunk * CB

            @pl.when(chunk + 1 < NCH)
            def _a():
                wait_idx(osl, base + CB)
                issue_gathers(osl)

            wait_gathers(sl)
            pltpu.sync_copy(eet.at[c, pl.ds(base, CB)], eev)

            @pl.when(chunk > 0)
            def _e():
                pltpu.make_async_copy(stage, acc.at[dsts], ssc).wait()

            @pl.loop(0, CB // 16)
            def _cp(j):
                dsts[pl.ds(j * 16, 16)] = dstv[sl][pl.ds(j * 16, 16)]

            @pl.when(chunk + 2 < NCH)
            def _g():
                issue_idx(sl, base + 2 * CB)

            @plsc.parallel_loop(0, CB, unroll=4)
            def _edge(j):
                p0 = (kvg[sl][j, pl.ds(0, 16)] * qg[sl][j, pl.ds(0, 16)]
                      + kvg[sl][j, pl.ds(16, 16)] * qg[sl][j, pl.ds(16, 16)])
                s0 = jnp.sum(p0) * INV_SQRT_DK
                p1 = (kvg[sl][j, pl.ds(32, 16)] * qg[sl][j, pl.ds(32, 16)]
                      + kvg[sl][j, pl.ds(48, 16)] * qg[sl][j, pl.ds(48, 16)])
                s1 = jnp.sum(p1) * INV_SQRT_DK
                for t in range(4):
                    sc_ = s0 if t < 2 else s1
                    a = jnp.exp(sc_ * (1.0 + eev[j, pl.ds(t * 16, 16)]))
                    stage[j, pl.ds(t * 16, 16)] = a
                    stage[j, pl.ds(64 + t * 16, 16)] = (
                        kvg[sl][j, pl.ds(64 + t * 16, 16)] * a)

            pltpu.async_copy(stage, acc.at[dsts], ssc, add=True)

    pltpu.make_async_copy(stage, acc.at[dsts], ssc).wait()
    plsc.subcore_barrier()

    r0 = s_ * NROW
    pltpu.sync_copy(acc.at[pl.ds(r0, NROW)], acc_hbm.at[c, pl.ds(r0, NROW)])


def _edge_stage_sc(KV2, Q2, Ee2, src, dst):
    mesh = plsc.VectorSubcoreMesh(core_axis_name="c", subcore_axis_name="s")
    cp = pltpu.CompilerParams()
    if "needs_layout_passes" in pltpu.CompilerParams.__dataclass_fields__:
        cp = dataclasses.replace(cp, needs_layout_passes=False)
    if "use_tc_tiling_on_sc" in pltpu.CompilerParams.__dataclass_fields__:
        cp = dataclasses.replace(cp, use_tc_tiling_on_sc=False)
    run = pl.kernel(
        _edge_body,
        compiler_params=cp,
        out_type=jax.ShapeDtypeStruct((2, NP, 128), jnp.float32),
        mesh=mesh,
        scratch_types=[
            pltpu.VMEM((CB,), jnp.int32),
            pltpu.VMEM((CB,), jnp.int32),
            pltpu.VMEM((CB,), jnp.int32),
            pltpu.VMEM((CB,), jnp.int32),
            pltpu.VMEM((CB,), jnp.int32),
            pltpu.VMEM((CB, 128), jnp.float32),
            pltpu.VMEM((CB, 128), jnp.float32),
            pltpu.VMEM((CB, 64), jnp.float32),
            pltpu.VMEM((CB, 64), jnp.float32),
            pltpu.VMEM((CB, 64), jnp.float32),
            pltpu.VMEM((CB, 128), jnp.float32),
            pltpu.VMEM_SHARED((NP, 128), jnp.float32),
            pltpu.SemaphoreType.DMA,
            pltpu.SemaphoreType.DMA,
            pltpu.SemaphoreType.DMA,
            pltpu.SemaphoreType.DMA,
            pltpu.SemaphoreType.DMA,
            pltpu.SemaphoreType.DMA,
            pltpu.SemaphoreType.DMA,
            pltpu.SemaphoreType.DMA,
            pltpu.SemaphoreType.DMA,
        ],
    )
    return run(KV2, Q2, Ee2, src, dst)


def _layer_norm(x, g, b):
    mu = jnp.mean(x, axis=-1, keepdims=True)
    xc = x - mu
    var = jnp.mean(xc * xc, axis=-1, keepdims=True)
    return xc * jax.lax.rsqrt(var + 1e-5) * g + b


def _epi_body(h_ref, a_ref, wlin_ref, blin_ref, g1_ref, b1_ref,
              wf1_ref, bf1_ref, wf2_ref, bf2_ref, g2_ref, b2_ref, o_ref):
    z = jnp.concatenate([a_ref[0][:, :64], a_ref[1][:, :64]], axis=-1)
    s = jnp.concatenate([a_ref[0][:, 64:], a_ref[1][:, 64:]], axis=-1)
    h_out = s / jnp.maximum(z, 1e-6)
    hb = h_ref[...]
    hr = hb + jnp.dot(h_out, wlin_ref[...], preferred_element_type=jnp.float32) + blin_ref[...]
    hr = _layer_norm(hr, g1_ref[...], b1_ref[...])
    t = jnp.dot(hr, wf1_ref[...], preferred_element_type=jnp.float32) + bf1_ref[...]
    t = jnp.maximum(t, 0.0)
    h2 = jnp.dot(t, wf2_ref[...], preferred_element_type=jnp.float32) + bf2_ref[...]
    o_ref[...] = _layer_norm(hr + h2, g2_ref[...], b2_ref[...])


def _epilogue(h, A2, Wlin, blin, g1, b1, Wf1, bf1, Wf2, bf2, g2, b2):
    full = lambda r, c: pl.BlockSpec((r, c), lambda i: (0, 0))
    vec = lambda c: pl.BlockSpec((c,), lambda i: (0,))
    return pl.pallas_call(
        _epi_body,
        grid=(N // BN,),
        in_specs=[
            pl.BlockSpec((BN, D), lambda i: (i, 0)),
            pl.BlockSpec((2, BN, 128), lambda i: (0, i, 0)),
            full(D, D), vec(D), vec(D), vec(D),
            full(D, 2 * D), vec(2 * D), full(2 * D, D), vec(D),
            vec(D), vec(D),
        ],
        out_specs=pl.BlockSpec((BN, D), lambda i: (i, 0)),
        out_shape=jax.ShapeDtypeStruct((N, D), jnp.float32),
    )(h, A2, Wlin, blin, g1, b1, Wf1, bf1, Wf2, bf2, g2, b2)


def kernel(h, e, edge_index, Wq, Wk, Wv, We, Wlin, blin, g1, b1,
           Wf1, bf1, Wf2, bf2, g2, b2):
    src = edge_index[0]
    dst = edge_index[1]
    Q2, KV2 = _qkv(h, Wq, Wk, Wv)
    Ee2 = _ee(e, We)
    A2 = _edge_stage_sc(KV2, Q2, Ee2, src, dst)
    return _epilogue(h, A2, Wlin, blin, g1, b1, Wf1, bf1, Wf2, bf2, g2, b2)


# speedup vs baseline: 71.2615x; 1.0586x over previous
import dataclasses
import functools
import math

import jax
import jax.numpy as jnp
from jax import lax
from jax.experimental import pallas as pl
from jax.experimental.pallas import tpu as pltpu
from jax.experimental.pallas import tpu_sc as plsc

N = 10000
NE = 320000
D = 128
H = 4
DK = D // H
INV_SQRT_DK = 1.0 / math.sqrt(DK)

BN = 1000
BE = 4000

NSC = 2
NSUB = 16
EPT = NE // NSUB
CB = 80
NCH = EPT // CB
NP = 10240
NROW = NP // NSUB


def _qkv_body(h_ref, wq_ref, wk_ref, wv_ref, q_ref, kv_ref):
    hb = h_ref[...].astype(jnp.bfloat16)
    qb = jnp.dot(hb, wq_ref[...].astype(jnp.bfloat16),
                 preferred_element_type=jnp.float32)
    kb = jnp.dot(hb, wk_ref[...].astype(jnp.bfloat16),
                 preferred_element_type=jnp.float32)
    vb = jnp.dot(hb, wv_ref[...].astype(jnp.bfloat16),
                 preferred_element_type=jnp.float32)
    q_ref[0] = qb[:, :64]
    q_ref[1] = qb[:, 64:]
    kv_ref[0] = jnp.concatenate([kb[:, :64], vb[:, :64]], axis=1)
    kv_ref[1] = jnp.concatenate([kb[:, 64:], vb[:, 64:]], axis=1)


def _qkv(h, Wq, Wk, Wv):
    wspec = lambda: pl.BlockSpec((D, D), lambda i: (0, 0))
    return pl.pallas_call(
        _qkv_body,
        grid=(N // BN,),
        in_specs=[pl.BlockSpec((BN, D), lambda i: (i, 0)),
                  wspec(), wspec(), wspec()],
        out_specs=[pl.BlockSpec((2, BN, 64), lambda i: (0, i, 0)),
                   pl.BlockSpec((2, BN, 128), lambda i: (0, i, 0))],
        out_shape=[jax.ShapeDtypeStruct((2, N, 64), jnp.float32),
                   jax.ShapeDtypeStruct((2, N, 128), jnp.float32)],
    )(h, Wq, Wk, Wv)


def _ee_body(e_ref, we_ref, o_ref):
    eb = jnp.dot(e_ref[...].astype(jnp.bfloat16),
                 we_ref[...].astype(jnp.bfloat16),
                 preferred_element_type=jnp.float32)
    o_ref[0] = eb[:, :64]
    o_ref[1] = eb[:, 64:]


def _ee(e, We):
    return pl.pallas_call(
        _ee_body,
        grid=(NE // BE,),
        in_specs=[pl.BlockSpec((BE, D), lambda i: (i, 0)),
                  pl.BlockSpec((D, D), lambda i: (0, 0))],
        out_specs=pl.BlockSpec((2, BE, 64), lambda i: (0, i, 0)),
        out_shape=jax.ShapeDtypeStruct((2, NE, 64), jnp.float32),
    )(e, We)


def _edge_body(kvt, qt, eet, src_hbm, dst_hbm, acc_hbm,
               srcv0, srcv1, dstv0, dstv1, dsts,
               kvg0, kvg1, qg0, qg1, eev, stage, acc,
               skv0, skv1, sq0, sq1, ssi0, ssi1, sdi0, sdi1, ssc):
    c = lax.axis_index("c")
    s_ = lax.axis_index("s")
    base0 = s_ * EPT

    srcv = (srcv0, srcv1)
    dstv = (dstv0, dstv1)
    kvg = (kvg0, kvg1)
    qg = (qg0, qg1)
    skv = (skv0, skv1)
    sq = (sq0, sq1)
    ssi = (ssi0, ssi1)
    sdi = (sdi0, sdi1)

    def issue_gathers(sl):
        pltpu.async_copy(kvt.at[c].at[srcv[sl]], kvg[sl], skv[sl])
        pltpu.async_copy(qt.at[c].at[dstv[sl]], qg[sl], sq[sl])

    def wait_gathers(sl):
        pltpu.make_async_copy(kvt.at[c].at[srcv[sl]], kvg[sl], skv[sl]).wait()
        pltpu.make_async_copy(qt.at[c].at[dstv[sl]], qg[sl], sq[sl]).wait()

    def issue_idx(sl, base):
        pltpu.async_copy(src_hbm.at[pl.ds(base, CB)], srcv[sl], ssi[sl])
        pltpu.async_copy(dst_hbm.at[pl.ds(base, CB)], dstv[sl], sdi[sl])

    def wait_idx(sl, base):
        pltpu.make_async_copy(src_hbm.at[pl.ds(base, CB)], srcv[sl], ssi[sl]).wait()
        pltpu.make_async_copy(dst_hbm.at[pl.ds(base, CB)], dstv[sl], sdi[sl]).wait()

    pltpu.sync_copy(src_hbm.at[pl.ds(base0, CB)], srcv0)
    pltpu.sync_copy(dst_hbm.at[pl.ds(base0, CB)], dstv0)
    issue_gathers(0)
    issue_idx(1, base0 + CB)

    zvec = jnp.zeros((16,), jnp.float32)

    @pl.loop(0, CB)
    def _zrow(i):
        @pl.loop(0, 8)
        def _zcol(j):
            stage[i, pl.ds(j * 16, 16)] = zvec

    @pl.loop(0, NROW // CB)
    def _zcp(t):
        pltpu.sync_copy(stage, acc.at[pl.ds(s_ * NROW + t * CB, CB)])

    plsc.subcore_barrier()

    @pl.loop(0, NCH, step=2)
    def _pair(ii):
        for sl in (0, 1):
            osl = 1 - sl
            chunk = ii + sl
            base = base0 + chunk * CB

            @pl.when(chunk + 1 < NCH)
            def _a():
                wait_idx(osl, base + CB)
                issue_gathers(osl)

            wait_gathers(sl)
            pltpu.sync_copy(eet.at[c, pl.ds(base, CB)], eev)

            @pl.when(chunk > 0)
            def _e():
                pltpu.make_async_copy(stage, acc.at[dsts], ssc).wait()

            @pl.loop(0, CB // 16)
            def _cp(j):
                dsts[pl.ds(j * 16, 16)] = dstv[sl][pl.ds(j * 16, 16)]

            @pl.when(chunk + 2 < NCH)
            def _g():
                issue_idx(sl, base + 2 * CB)

            @plsc.parallel_loop(0, CB, unroll=8)
            def _edge(j):
                p0 = (kvg[sl][j, pl.ds(0, 16)] * qg[sl][j, pl.ds(0, 16)]
                      + kvg[sl][j, pl.ds(16, 16)] * qg[sl][j, pl.ds(16, 16)])
                s0 = jnp.sum(p0) * INV_SQRT_DK
                p1 = (kvg[sl][j, pl.ds(32, 16)] * qg[sl][j, pl.ds(32, 16)]
                      + kvg[sl][j, pl.ds(48, 16)] * qg[sl][j, pl.ds(48, 16)])
                s1 = jnp.sum(p1) * INV_SQRT_DK
                for t in range(4):
                    sc_ = s0 if t < 2 else s1
                    a = jnp.exp(sc_ * (1.0 + eev[j, pl.ds(t * 16, 16)]))
                    stage[j, pl.ds(t * 16, 16)] = a
                    stage[j, pl.ds(64 + t * 16, 16)] = (
                        kvg[sl][j, pl.ds(64 + t * 16, 16)] * a)

            pltpu.async_copy(stage, acc.at[dsts], ssc, add=True)

    pltpu.make_async_copy(stage, acc.at[dsts], ssc).wait()
    plsc.subcore_barrier()

    r0 = s_ * NROW
    pltpu.sync_copy(acc.at[pl.ds(r0, NROW)], acc_hbm.at[c, pl.ds(r0, NROW)])


def _edge_stage_sc(KV2, Q2, Ee2, src, dst):
    mesh = plsc.VectorSubcoreMesh(core_axis_name="c", subcore_axis_name="s")
    cp = pltpu.CompilerParams()
    if "needs_layout_passes" in pltpu.CompilerParams.__dataclass_fields__:
        cp = dataclasses.replace(cp, needs_layout_passes=False)
    if "use_tc_tiling_on_sc" in pltpu.CompilerParams.__dataclass_fields__:
        cp = dataclasses.replace(cp, use_tc_tiling_on_sc=False)
    run = pl.kernel(
        _edge_body,
        compiler_params=cp,
        out_type=jax.ShapeDtypeStruct((2, NP, 128), jnp.float32),
        mesh=mesh,
        scratch_types=[
            pltpu.VMEM((CB,), jnp.int32),
            pltpu.VMEM((CB,), jnp.int32),
            pltpu.VMEM((CB,), jnp.int32),
            pltpu.VMEM((CB,), jnp.int32),
            pltpu.VMEM((CB,), jnp.int32),
            pltpu.VMEM((CB, 128), jnp.float32),
            pltpu.VMEM((CB, 128), jnp.float32),
            pltpu.VMEM((CB, 64), jnp.float32),
            pltpu.VMEM((CB, 64), jnp.float32),
            pltpu.VMEM((CB, 64), jnp.float32),
            pltpu.VMEM((CB, 128), jnp.float32),
            pltpu.VMEM_SHARED((NP, 128), jnp.float32),
            pltpu.SemaphoreType.DMA,
            pltpu.SemaphoreType.DMA,
            pltpu.SemaphoreType.DMA,
            pltpu.SemaphoreType.DMA,
            pltpu.SemaphoreType.DMA,
            pltpu.SemaphoreType.DMA,
            pltpu.SemaphoreType.DMA,
            pltpu.SemaphoreType.DMA,
            pltpu.SemaphoreType.DMA,
        ],
    )
    return run(KV2, Q2, Ee2, src, dst)


def _layer_norm(x, g, b):
    mu = jnp.mean(x, axis=-1, keepdims=True)
    xc = x - mu
    var = jnp.mean(xc * xc, axis=-1, keepdims=True)
    return xc * jax.lax.rsqrt(var + 1e-5) * g + b


def _epi_body(h_ref, a_ref, wlin_ref, blin_ref, g1_ref, b1_ref,
              wf1_ref, bf1_ref, wf2_ref, bf2_ref, g2_ref, b2_ref, o_ref):
    z = jnp.concatenate([a_ref[0][:, :64], a_ref[1][:, :64]], axis=-1)
    s = jnp.concatenate([a_ref[0][:, 64:], a_ref[1][:, 64:]], axis=-1)
    h_out = s / jnp.maximum(z, 1e-6)
    hb = h_ref[...]
    hr = hb + jnp.dot(h_out, wlin_ref[...], preferred_element_type=jnp.float32) + blin_ref[...]
    hr = _layer_norm(hr, g1_ref[...], b1_ref[...])
    t = jnp.dot(hr, wf1_ref[...], preferred_element_type=jnp.float32) + bf1_ref[...]
    t = jnp.maximum(t, 0.0)
    h2 = jnp.dot(t, wf2_ref[...], preferred_element_type=jnp.float32) + bf2_ref[...]
    o_ref[...] = _layer_norm(hr + h2, g2_ref[...], b2_ref[...])


def _epilogue(h, A2, Wlin, blin, g1, b1, Wf1, bf1, Wf2, bf2, g2, b2):
    full = lambda r, c: pl.BlockSpec((r, c), lambda i: (0, 0))
    vec = lambda c: pl.BlockSpec((c,), lambda i: (0,))
    return pl.pallas_call(
        _epi_body,
        grid=(N // BN,),
        in_specs=[
            pl.BlockSpec((BN, D), lambda i: (i, 0)),
            pl.BlockSpec((2, BN, 128), lambda i: (0, i, 0)),
            full(D, D), vec(D), vec(D), vec(D),
            full(D, 2 * D), vec(2 * D), full(2 * D, D), vec(D),
            vec(D), vec(D),
        ],
        out_specs=pl.BlockSpec((BN, D), lambda i: (i, 0)),
        out_shape=jax.ShapeDtypeStruct((N, D), jnp.float32),
    )(h, A2, Wlin, blin, g1, b1, Wf1, bf1, Wf2, bf2, g2, b2)


def kernel(h, e, edge_index, Wq, Wk, Wv, We, Wlin, blin, g1, b1,
           Wf1, bf1, Wf2, bf2, g2, b2):
    src = edge_index[0]
    dst = edge_index[1]
    Q2, KV2 = _qkv(h, Wq, Wk, Wv)
    Ee2 = _ee(e, We)
    A2 = _edge_stage_sc(KV2, Q2, Ee2, src, dst)
    return _epilogue(h, A2, Wlin, blin, g1, b1, Wf1, bf1, Wf2, bf2, g2, b2)


# speedup vs baseline: 72.1148x; 1.0120x over previous
import dataclasses
import functools
import math

import jax
import jax.numpy as jnp
import numpy as np
from jax import lax
from jax.experimental import pallas as pl
from jax.experimental.pallas import tpu as pltpu
from jax.experimental.pallas import tpu_sc as plsc

N = 10000
NE = 320000
D = 128
H = 4
DK = D // H
INV_SQRT_DK = 1.0 / math.sqrt(DK)

BN = 1000
BE = 4000

NSC = 2
NSUB = 16
EPT = NE // NSUB
CB = 80
NCH = EPT // CB
NP = 10240
NROW = NP // NSUB


def _qkv_body(h_ref, wq_ref, wk_ref, wv_ref, q_ref, kv_ref):
    hb = h_ref[...].astype(jnp.bfloat16)
    qb = jnp.dot(hb, wq_ref[...].astype(jnp.bfloat16),
                 preferred_element_type=jnp.float32)
    kb = jnp.dot(hb, wk_ref[...].astype(jnp.bfloat16),
                 preferred_element_type=jnp.float32)
    vb = jnp.dot(hb, wv_ref[...].astype(jnp.bfloat16),
                 preferred_element_type=jnp.float32)
    q_ref[0] = qb[:, :64]
    q_ref[1] = qb[:, 64:]
    kv_ref[0] = jnp.concatenate([kb[:, :64], vb[:, :64]], axis=1)
    kv_ref[1] = jnp.concatenate([kb[:, 64:], vb[:, 64:]], axis=1)


def _qkv(h, Wq, Wk, Wv):
    wspec = lambda: pl.BlockSpec((D, D), lambda i: (0, 0))
    return pl.pallas_call(
        _qkv_body,
        grid=(N // BN,),
        in_specs=[pl.BlockSpec((BN, D), lambda i: (i, 0)),
                  wspec(), wspec(), wspec()],
        out_specs=[pl.BlockSpec((2, BN, 64), lambda i: (0, i, 0)),
                   pl.BlockSpec((2, BN, 128), lambda i: (0, i, 0))],
        out_shape=[jax.ShapeDtypeStruct((2, N, 64), jnp.float32),
                   jax.ShapeDtypeStruct((2, N, 128), jnp.float32)],
    )(h, Wq, Wk, Wv)


def _ee_body(e_ref, we_ref, o_ref):
    eb = jnp.dot(e_ref[...].astype(jnp.bfloat16),
                 we_ref[...].astype(jnp.bfloat16),
                 preferred_element_type=jnp.float32)
    eb16 = eb.astype(jnp.bfloat16)
    o_ref[0] = eb16[:, :64]
    o_ref[1] = eb16[:, 64:]


def _ee(e, We):
    return pl.pallas_call(
        _ee_body,
        grid=(NE // BE,),
        in_specs=[pl.BlockSpec((BE, D), lambda i: (i, 0)),
                  pl.BlockSpec((D, D), lambda i: (0, 0))],
        out_specs=pl.BlockSpec((2, BE, 64), lambda i: (0, i, 0)),
        out_shape=jax.ShapeDtypeStruct((2, NE, 64), jnp.bfloat16),
    )(e, We)


def _edge_body(kvt, qt, eet, src_hbm, dst_hbm, acc_hbm,
               srcv0, srcv1, dstv0, dstv1, dsts,
               kvg0, kvg1, qg0, qg1, eev, stage, acc,
               skv0, skv1, sq0, sq1, ssi0, ssi1, sdi0, sdi1, ssc):
    c = lax.axis_index("c")
    s_ = lax.axis_index("s")
    base0 = s_ * EPT

    srcv = (srcv0, srcv1)
    dstv = (dstv0, dstv1)
    kvg = (kvg0, kvg1)
    qg = (qg0, qg1)
    skv = (skv0, skv1)
    sq = (sq0, sq1)
    ssi = (ssi0, ssi1)
    sdi = (sdi0, sdi1)

    def issue_gathers(sl):
        pltpu.async_copy(kvt.at[c].at[srcv[sl]], kvg[sl], skv[sl])
        pltpu.async_copy(qt.at[c].at[dstv[sl]], qg[sl], sq[sl])

    def wait_gathers(sl):
        pltpu.make_async_copy(kvt.at[c].at[srcv[sl]], kvg[sl], skv[sl]).wait()
        pltpu.make_async_copy(qt.at[c].at[dstv[sl]], qg[sl], sq[sl]).wait()

    def issue_idx(sl, base):
        pltpu.async_copy(src_hbm.at[pl.ds(base, CB)], srcv[sl], ssi[sl])
        pltpu.async_copy(dst_hbm.at[pl.ds(base, CB)], dstv[sl], sdi[sl])

    def wait_idx(sl, base):
        pltpu.make_async_copy(src_hbm.at[pl.ds(base, CB)], srcv[sl], ssi[sl]).wait()
        pltpu.make_async_copy(dst_hbm.at[pl.ds(base, CB)], dstv[sl], sdi[sl]).wait()

    pltpu.sync_copy(src_hbm.at[pl.ds(base0, CB)], srcv0)
    pltpu.sync_copy(dst_hbm.at[pl.ds(base0, CB)], dstv0)
    issue_gathers(0)
    issue_idx(1, base0 + CB)

    zvec = jnp.zeros((16,), jnp.float32)

    @pl.loop(0, CB)
    def _zrow(i):
        @pl.loop(0, 8)
        def _zcol(j):
            stage[i, pl.ds(j * 16, 16)] = zvec

    @pl.loop(0, NROW // CB)
    def _zcp(t):
        pltpu.sync_copy(stage, acc.at[pl.ds(s_ * NROW + t * CB, CB)])

    plsc.subcore_barrier()

    @pl.loop(0, NCH, step=2)
    def _pair(ii):
        for sl in (0, 1):
            osl = 1 - sl
            chunk = ii + sl
            base = base0 + chunk * CB

            @pl.when(chunk + 1 < NCH)
            def _a():
                wait_idx(osl, base + CB)
                issue_gathers(osl)

            wait_gathers(sl)
            pltpu.sync_copy(eet.at[c, pl.ds(base, CB)], eev)

            @pl.when(chunk > 0)
            def _e():
                pltpu.make_async_copy(stage, acc.at[dsts], ssc).wait()

            @pl.loop(0, CB // 16)
            def _cp(j):
                dsts[pl.ds(j * 16, 16)] = dstv[sl][pl.ds(j * 16, 16)]

            @pl.when(chunk + 2 < NCH)
            def _g():
                issue_idx(sl, base + 2 * CB)

            @plsc.parallel_loop(0, CB, unroll=8)
            def _edge(j):
                p0 = (kvg[sl][j, pl.ds(0, 16)] * qg[sl][j, pl.ds(0, 16)]
                      + kvg[sl][j, pl.ds(16, 16)] * qg[sl][j, pl.ds(16, 16)])
                s0 = jnp.sum(p0) * INV_SQRT_DK
                p1 = (kvg[sl][j, pl.ds(32, 16)] * qg[sl][j, pl.ds(32, 16)]
                      + kvg[sl][j, pl.ds(48, 16)] * qg[sl][j, pl.ds(48, 16)])
                s1 = jnp.sum(p1) * INV_SQRT_DK
                for t in range(2):
                    sc_ = s0 if t == 0 else s1
                    xi = plsc.bitcast(eev[j, pl.ds(t * 32, 32)], jnp.int32)
                    elo = plsc.bitcast(jnp.left_shift(xi, 16), jnp.float32)
                    ehi = plsc.bitcast(jnp.bitwise_and(xi, jnp.int32(-65536)),
                                       jnp.float32)
                    a0 = jnp.exp(sc_ * (1.0 + elo))
                    a1 = jnp.exp(sc_ * (1.0 + ehi))
                    stage[j, pl.ds(t * 32, 16)] = a0
                    stage[j, pl.ds(t * 32 + 16, 16)] = a1
                    stage[j, pl.ds(64 + t * 32, 16)] = (
                        kvg[sl][j, pl.ds(64 + t * 32, 16)] * a0)
                    stage[j, pl.ds(64 + t * 32 + 16, 16)] = (
                        kvg[sl][j, pl.ds(64 + t * 32 + 16, 16)] * a1)

            pltpu.async_copy(stage, acc.at[dsts], ssc, add=True)

    pltpu.make_async_copy(stage, acc.at[dsts], ssc).wait()
    plsc.subcore_barrier()

    r0 = s_ * NROW
    pltpu.sync_copy(acc.at[pl.ds(r0, NROW)], acc_hbm.at[c, pl.ds(r0, NROW)])


def _edge_stage_sc(KV2, Q2, Ee2, src, dst):
    mesh = plsc.VectorSubcoreMesh(core_axis_name="c", subcore_axis_name="s")
    cp = pltpu.CompilerParams()
    if "needs_layout_passes" in pltpu.CompilerParams.__dataclass_fields__:
        cp = dataclasses.replace(cp, needs_layout_passes=False)
    if "use_tc_tiling_on_sc" in pltpu.CompilerParams.__dataclass_fields__:
        cp = dataclasses.replace(cp, use_tc_tiling_on_sc=False)
    run = pl.kernel(
        _edge_body,
        compiler_params=cp,
        out_type=jax.ShapeDtypeStruct((2, NP, 128), jnp.float32),
        mesh=mesh,
        scratch_types=[
            pltpu.VMEM((CB,), jnp.int32),
            pltpu.VMEM((CB,), jnp.int32),
            pltpu.VMEM((CB,), jnp.int32),
            pltpu.VMEM((CB,), jnp.int32),
            pltpu.VMEM((CB,), jnp.int32),
            pltpu.VMEM((CB, 128), jnp.float32),
            pltpu.VMEM((CB, 128), jnp.float32),
            pltpu.VMEM((CB, 64), jnp.float32),
            pltpu.VMEM((CB, 64), jnp.float32),
            pltpu.VMEM((CB, 64), jnp.bfloat16),
            pltpu.VMEM((CB, 128), jnp.float32),
            pltpu.VMEM_SHARED((NP, 128), jnp.float32),
            pltpu.SemaphoreType.DMA,
            pltpu.SemaphoreType.DMA,
            pltpu.SemaphoreType.DMA,
            pltpu.SemaphoreType.DMA,
            pltpu.SemaphoreType.DMA,
            pltpu.SemaphoreType.DMA,
            pltpu.SemaphoreType.DMA,
            pltpu.SemaphoreType.DMA,
            pltpu.SemaphoreType.DMA,
        ],
    )
    return run(KV2, Q2, Ee2, src, dst)


def _layer_norm(x, g, b):
    mu = jnp.mean(x, axis=-1, keepdims=True)
    xc = x - mu
    var = jnp.mean(xc * xc, axis=-1, keepdims=True)
    return xc * jax.lax.rsqrt(var + 1e-5) * g + b


def _epi_body(h_ref, a_ref, wlin_ref, blin_ref, g1_ref, b1_ref,
              wf1_ref, bf1_ref, wf2_ref, bf2_ref, g2_ref, b2_ref, o_ref):
    z = jnp.concatenate([a_ref[0][:, :64], a_ref[1][:, :64]], axis=-1)
    s = jnp.concatenate([a_ref[0][:, 64:], a_ref[1][:, 64:]], axis=-1)
    h_out = s / jnp.maximum(z, 1e-6)
    hb = h_ref[...]
    hr = hb + jnp.dot(h_out, wlin_ref[...], preferred_element_type=jnp.float32) + blin_ref[...]
    hr = _layer_norm(hr, g1_ref[...], b1_ref[...])
    t = jnp.dot(hr, wf1_ref[...], preferred_element_type=jnp.float32) + bf1_ref[...]
    t = jnp.maximum(t, 0.0)
    h2 = jnp.dot(t, wf2_ref[...], preferred_element_type=jnp.float32) + bf2_ref[...]
    o_ref[...] = _layer_norm(hr + h2, g2_ref[...], b2_ref[...])


def _epilogue(h, A2, Wlin, blin, g1, b1, Wf1, bf1, Wf2, bf2, g2, b2):
    full = lambda r, c: pl.BlockSpec((r, c), lambda i: (0, 0))
    vec = lambda c: pl.BlockSpec((c,), lambda i: (0,))
    return pl.pallas_call(
        _epi_body,
        grid=(N // BN,),
        in_specs=[
            pl.BlockSpec((BN, D), lambda i: (i, 0)),
            pl.BlockSpec((2, BN, 128), lambda i: (0, i, 0)),
            full(D, D), vec(D), vec(D), vec(D),
            full(D, 2 * D), vec(2 * D), full(2 * D, D), vec(D),
            vec(D), vec(D),
        ],
        out_specs=pl.BlockSpec((BN, D), lambda i: (i, 0)),
        out_shape=jax.ShapeDtypeStruct((N, D), jnp.float32),
    )(h, A2, Wlin, blin, g1, b1, Wf1, bf1, Wf2, bf2, g2, b2)


_EE_PERM = np.asarray(
    [32 * (j // 32) + (j % 2) * 16 + (j // 2) % 16 for j in range(128)],
    dtype=np.int32)


def kernel(h, e, edge_index, Wq, Wk, Wv, We, Wlin, blin, g1, b1,
           Wf1, bf1, Wf2, bf2, g2, b2):
    src = edge_index[0]
    dst = edge_index[1]
    We = We[:, _EE_PERM]
    Q2, KV2 = _qkv(h, Wq, Wk, Wv)
    Ee2 = _ee(e, We)
    A2 = _edge_stage_sc(KV2, Q2, Ee2, src, dst)
    return _epilogue(h, A2, Wlin, blin, g1, b1, Wf1, bf1, Wf2, bf2, g2, b2)
